# baseline (speedup 1.0000x reference)
import functools

import jax
import jax.numpy as jnp
from jax import lax
from jax.experimental import pallas as pl
from jax.experimental.pallas import tpu as pltpu
from jax.experimental.pallas import tpu_sc as plsc

T = 2048
D = 1024
E = 8
H = 2048
C = 512
S = E * C
SENT = S
NW = 32
SLOTS_PER_W = S // NW
TOK_PER_W = T // NW


def _router_body(x_ref, wg_ref, logits_ref, slots_ref, gates_ref, aux_ref):
    x = x_ref[...]
    wg = wg_ref[...]
    logits = jnp.dot(x, wg, preferred_element_type=jnp.float32)
    logits_ref[...] = logits

    m = jnp.max(logits, axis=-1, keepdims=True)
    ex = jnp.exp(logits - m)
    probs = ex / jnp.sum(ex, axis=-1, keepdims=True)

    colid = lax.broadcasted_iota(jnp.int32, (T, E), 1)
    g1 = jnp.max(probs, axis=-1, keepdims=True)
    idx1 = jnp.min(jnp.where(probs == g1, colid, E), axis=-1, keepdims=True)
    mask1 = (colid == idx1).astype(jnp.float32)
    probs2 = probs * (1.0 - mask1)
    g2 = jnp.max(probs2, axis=-1, keepdims=True)
    idx2 = jnp.min(jnp.where(probs2 == g2, colid, E), axis=-1, keepdims=True)
    mask2 = (colid == idx2).astype(jnp.float32)

    denom = g1 + g2 + 1e-9
    g1n = g1 / denom
    g2n = g2 / denom

    def incl_cumsum0(a):
        s = a
        shift = 1
        while shift < T:
            z = jnp.zeros((shift, E), jnp.float32)
            s = s + jnp.concatenate([z, s[: T - shift]], axis=0)
            shift *= 2
        return s

    cs1 = incl_cumsum0(mask1)
    pos1 = cs1 - mask1
    pos1_tok = jnp.sum(pos1 * mask1, axis=-1, keepdims=True)
    count1 = cs1[T - 1 :, :]
    cs2 = incl_cumsum0(mask2)
    pos2 = cs2 - mask2 + count1
    pos2_tok = jnp.sum(pos2 * mask2, axis=-1, keepdims=True)

    keep1 = (pos1_tok < C).astype(jnp.float32)
    keep2 = (pos2_tok < C).astype(jnp.float32)

    slot1 = idx1 * C + pos1_tok.astype(jnp.int32)
    slot2 = idx2 * C + pos2_tok.astype(jnp.int32)
    s1 = jnp.where(keep1 > 0.0, slot1, SENT)
    s2 = jnp.where(keep2 > 0.0, slot2, SENT)
    g1k = g1n * keep1
    g2k = g2n * keep2

    slots_ref[...] = (jnp.where(colid == 0, s1, 0)
                      + jnp.where(colid == 1, s2, 0))
    gates_ref[...] = (jnp.where(colid == 0, g1k, 0.0)
                      + jnp.where(colid == 1, g2k, 0.0))

    f = jnp.mean(mask1 * keep1, axis=0, keepdims=True)
    p = jnp.mean(probs, axis=0, keepdims=True)
    aux_ref[...] = (E * jnp.sum(f * p)).reshape(1, 1)


def _router(x, wg):
    return pl.pallas_call(
        _router_body,
        out_shape=(
            jax.ShapeDtypeStruct((T, E), jnp.float32),
            jax.ShapeDtypeStruct((T, E), jnp.int32),
            jax.ShapeDtypeStruct((T, E), jnp.float32),
            jax.ShapeDtypeStruct((1, 1), jnp.float32),
        ),
    )(x, wg)


def _dispatch_body(x_hbm, sl_hbm, g_hbm, ein_hbm, gsc_hbm,
                   sl_v, gv_v, tfs_v, gsc_v, idx_v, rows_v, sem):
    c = lax.axis_index("c")
    sc = lax.axis_index("s")
    wid = sc * 2 + c

    pltpu.sync_copy(sl_hbm, sl_v)
    pltpu.sync_copy(g_hbm, gv_v)

    iota = lax.iota(jnp.int32, 16)
    sent_tok = jnp.zeros((16,), jnp.int32)
    zf = jnp.zeros((16,), jnp.float32)

    def init_body(i, carry):
        tfs_v[pl.ds(i * 16, 16)] = sent_tok
        gsc_v[pl.ds(i * 16, 16)] = zf
        return carry

    lax.fori_loop(0, S // 16, init_body, 0)

    def scat_body(i, carry):
        sl = sl_v[pl.ds(i * 16, 16)]
        gg = gv_v[pl.ds(i * 16, 16)]
        tok = (iota + i * 16) & (T - 1)
        msk = sl < SENT
        sl_c = jnp.where(msk, sl, 0)
        plsc.store_scatter(tfs_v, [sl_c], tok, mask=msk)
        plsc.store_scatter(gsc_v, [sl_c], gg, mask=msk)
        return carry

    lax.fori_loop(0, S // 16, scat_body, 0)

    base = wid * SLOTS_PER_W
    for half in range(2):
        sb = base + half * 64
        for j in range(4):
            idx_v[pl.ds(j * 16, 16)] = tfs_v[pl.ds(sb + j * 16, 16)]
        pltpu.async_copy(x_hbm.at[idx_v], rows_v, sem).wait()
        pltpu.sync_copy(rows_v, ein_hbm.at[pl.ds(sb, 64)])

    pltpu.sync_copy(gsc_v.at[pl.ds(base, SLOTS_PER_W)],
                    gsc_hbm.at[pl.ds(base, SLOTS_PER_W)])


def _dispatch(x, s_all, g_all):
    mesh = plsc.VectorSubcoreMesh(core_axis_name="c", subcore_axis_name="s")
    return pl.kernel(
        _dispatch_body,
        out_type=(
            jax.ShapeDtypeStruct((S, D), jnp.float32),
            jax.ShapeDtypeStruct((S,), jnp.float32),
        ),
        mesh=mesh,
        compiler_params=pltpu.CompilerParams(needs_layout_passes=False),
        scratch_types=[
            pltpu.VMEM((S,), jnp.int32),
            pltpu.VMEM((S,), jnp.float32),
            pltpu.VMEM((S,), jnp.int32),
            pltpu.VMEM((S,), jnp.float32),
            pltpu.VMEM((64,), jnp.int32),
            pltpu.VMEM((64, D), jnp.float32),
            pltpu.SemaphoreType.DMA,
        ],
    )(x, s_all, g_all)


def _ffn_body(ein_ref, w1_ref, b1_ref, w2_ref, b2_ref, gsc_ref, out_ref):
    e = pl.program_id(0)

    @pl.when(e < E)
    def _compute():
        xb = ein_ref[...]
        h = jnp.dot(xb, w1_ref[0], preferred_element_type=jnp.float32)
        h = jax.nn.gelu(h + b1_ref[0])
        o = jnp.dot(h, w2_ref[0], preferred_element_type=jnp.float32)
        o = o + b2_ref[0]
        out_ref[...] = o * gsc_ref[...]

    @pl.when(e == E)
    def _zeros():
        out_ref[...] = jnp.zeros((C, D), jnp.float32)


def _ffn(ein, w1, b1, w2, b2, gsc):
    grid = (E + 1,)
    return pl.pallas_call(
        _ffn_body,
        grid=grid,
        in_specs=[
            pl.BlockSpec((C, D), lambda e: (jnp.minimum(e, E - 1), 0)),
            pl.BlockSpec((1, D, H), lambda e: (jnp.minimum(e, E - 1), 0, 0)),
            pl.BlockSpec((1, 1, H), lambda e: (jnp.minimum(e, E - 1), 0, 0)),
            pl.BlockSpec((1, H, D), lambda e: (jnp.minimum(e, E - 1), 0, 0)),
            pl.BlockSpec((1, 1, D), lambda e: (jnp.minimum(e, E - 1), 0, 0)),
            pl.BlockSpec((C, 1), lambda e: (jnp.minimum(e, E - 1), 0)),
        ],
        out_specs=pl.BlockSpec((C, D), lambda e: (e, 0)),
        out_shape=jax.ShapeDtypeStruct((S + C, D), jnp.float32),
    )(ein, w1, b1.reshape(E, 1, H), w2, b2.reshape(E, 1, D), gsc)


def _combine_body(eo_hbm, sidx_hbm, y_hbm, ia_v, ib_v, r1_v, r2_v, sa, sb):
    c = lax.axis_index("c")
    sc = lax.axis_index("s")
    wid = sc * 2 + c
    tb = wid * TOK_PER_W

    for half in range(2):
        t0 = tb + half * 32
        pltpu.sync_copy(sidx_hbm.at[pl.ds(t0, 32)], ia_v)
        pltpu.sync_copy(sidx_hbm.at[pl.ds(T + t0, 32)], ib_v)
        cpa = pltpu.async_copy(eo_hbm.at[ia_v], r1_v, sa)
        cpb = pltpu.async_copy(eo_hbm.at[ib_v], r2_v, sb)
        cpa.wait()
        cpb.wait()

        def add_body(t, carry):
            for k in range(D // 16):
                r1_v[t, pl.ds(k * 16, 16)] = (
                    r1_v[t, pl.ds(k * 16, 16)] + r2_v[t, pl.ds(k * 16, 16)])
            return carry

        lax.fori_loop(0, 32, add_body, 0)
        pltpu.sync_copy(r1_v, y_hbm.at[pl.ds(t0, 32)])


def _combine(eo, s_all):
    mesh = plsc.VectorSubcoreMesh(core_axis_name="c", subcore_axis_name="s")
    return pl.kernel(
        _combine_body,
        out_type=jax.ShapeDtypeStruct((T, D), jnp.float32),
        mesh=mesh,
        compiler_params=pltpu.CompilerParams(needs_layout_passes=False),
        scratch_types=[
            pltpu.VMEM((32,), jnp.int32),
            pltpu.VMEM((32,), jnp.int32),
            pltpu.VMEM((32, D), jnp.float32),
            pltpu.VMEM((32, D), jnp.float32),
            pltpu.SemaphoreType.DMA,
            pltpu.SemaphoreType.DMA,
        ],
    )(eo, s_all)


def kernel(x, Wg, W1, b1, W2, b2):
    logits, slots, gates, aux = _router(x, Wg)
    s_all = jnp.concatenate([slots[:, 0], slots[:, 1]])
    g_all = jnp.concatenate([gates[:, 0], gates[:, 1]])
    ein, gsc = _dispatch(x, s_all, g_all)
    eo = _ffn(ein, W1, b1, W2, b2, gsc.reshape(S, 1))
    y = _combine(eo, s_all)
    metrics = {"aux_loss": aux[0, 0], "router_logits": logits}
    return y, metrics
